# baseline (speedup 1.0000x reference)
import functools

import jax
import jax.numpy as jnp
from jax import lax
from jax.experimental import pallas as pl
from jax.experimental.pallas import tpu as pltpu
from jax.experimental.pallas import tpu_sc as plsc

B = 4096
T = 50
D = 768
O = 1024
SUB = 256
SPT = D // SUB
NSUB = T * SPT
NPAD = 152
G1 = 104
G2 = NPAD - G1
NW = 32
BPW = B // NW
IDXW = BPW * NPAD
FL = 8


def _sc_segment_sum(sidx_flat, emb3):
    mesh = plsc.VectorSubcoreMesh(core_axis_name="c", subcore_axis_name="s")

    @functools.partial(
        pl.kernel,
        out_type=jax.ShapeDtypeStruct((B * D,), jnp.float32),
        mesh=mesh,
        scratch_types=[
            pltpu.VMEM((NPAD, SUB), jnp.float32),
            pltpu.VMEM((NPAD, SUB), jnp.float32),
            pltpu.VMEM((FL * D,), jnp.float32),
            pltpu.VMEM((FL * D,), jnp.float32),
            pltpu.VMEM((4, G1), jnp.int32),
            pltpu.VMEM((4, G2), jnp.int32),
            pltpu.SemaphoreType.DMA,
            pltpu.SemaphoreType.DMA,
            pltpu.SemaphoreType.DMA,
            pltpu.SemaphoreType.DMA,
            pltpu.SemaphoreType.DMA,
            pltpu.SemaphoreType.DMA,
            pltpu.SemaphoreType.DMA,
            pltpu.SemaphoreType.DMA,
        ],
    )
    def sc_kernel(sidx_hbm, emb_hbm, out_hbm, rows0, rows1,
                  stage0, stage1, idxa, idxb,
                  sem0, sem1, osem0, osem1, isem0, isem1, isem2, isem3):
        wid = lax.axis_index("s") * 2 + lax.axis_index("c")
        rows = [rows0, rows1]
        sems = [sem0, sem1]
        stages = [stage0, stage1]
        osems = [osem0, osem1]
        isems = [isem0, isem1, isem2, isem3]
        base = wid * BPW

        def idx_copies(q, b):
            off = pl.multiple_of(wid * IDXW + b * NPAD, 8)
            return (
                pltpu.make_async_copy(
                    sidx_hbm.at[pl.ds(off, G1)], idxa.at[q], isems[q]),
                pltpu.make_async_copy(
                    sidx_hbm.at[pl.ds(off + G1, G2)], idxb.at[q], isems[q]),
            )

        def gather_copies(s, q):
            return (
                pltpu.make_async_copy(
                    emb_hbm.at[idxa.at[q]],
                    rows[s].at[pl.ds(0, G1)], sems[s]),
                pltpu.make_async_copy(
                    emb_hbm.at[idxb.at[q]],
                    rows[s].at[pl.ds(G1, G2)], sems[s]),
            )

        def out_copy(half, f):
            row_off = pl.multiple_of(base * D + f * (FL * D), 8)
            return pltpu.make_async_copy(
                stages[half], out_hbm.at[pl.ds(row_off, FL * D)],
                osems[half])

        for q in range(4):
            for cp in idx_copies(q, q):
                cp.start()
        for s in range(2):
            for cp in idx_copies(s, s):
                cp.wait()
            for cp in gather_copies(s, s):
                cp.start()

        def flush_pair(p, carry):
            for half in range(2):
                f = p * 2 + half
                stage = stages[half]

                @pl.when(f >= 2)
                def _(f=f, half=half):
                    out_copy(half, f).wait()

                for j in range(FL):
                    b = f * FL + j
                    s = j % 2
                    q = j % 4
                    for cp in gather_copies(s, q):
                        cp.wait()

                    for k in range(SPT):
                        def add_row(t, acc, k=k, s=s):
                            return tuple(
                                acc[v] + rows[s][t * SPT + k,
                                                 pl.ds(v * 16, 16)]
                                for v in range(SUB // 16))

                        acc = plsc.parallel_loop(
                            0, T, unroll=4,
                            carry=tuple(jnp.zeros((16,), jnp.float32)
                                        for _ in range(SUB // 16)))(add_row)
                        for v in range(SUB // 16):
                            stage[pl.ds(j * D + k * SUB + v * 16, 16)] \
                                = acc[v]

                    @pl.when(b + 4 < BPW)
                    def _(b=b, q=q):
                        for cp in idx_copies(q, b + 4):
                            cp.start()

                    @pl.when(b + 2 < BPW)
                    def _(b=b, s=s, j=j):
                        q2 = (j + 2) % 4
                        for cp in idx_copies(q2, b + 2):
                            cp.wait()
                        for cp in gather_copies(s, q2):
                            cp.start()

                out_copy(half, f).start()
            return carry

        lax.fori_loop(0, BPW // FL // 2, flush_pair, 0)

        nf = BPW // FL
        for half in range(2):
            out_copy(half, nf - 2 + half).wait()

    return sc_kernel(sidx_flat, emb3)


def _tc_proj_norm(zsum, W, b2d):
    blk = 256

    def tc_kernel(z_ref, w_ref, b_ref, o_ref):
        z = z_ref[...] * (1.0 / T)
        y = jnp.dot(z, w_ref[...], preferred_element_type=jnp.float32)
        y = y + b_ref[...]
        n = jnp.sqrt(jnp.sum(y * y, axis=1, keepdims=True))
        o_ref[...] = y / jnp.maximum(n, 1e-12)

    return pl.pallas_call(
        tc_kernel,
        grid=(B // blk,),
        in_specs=[
            pl.BlockSpec((blk, D), lambda i: (i, 0)),
            pl.BlockSpec((D, O), lambda i: (0, 0)),
            pl.BlockSpec((1, O), lambda i: (0, 0)),
        ],
        out_specs=pl.BlockSpec((blk, O), lambda i: (i, 0)),
        out_shape=jax.ShapeDtypeStruct((B, O), jnp.float32),
    )(zsum, W, b2d)


@jax.jit
def kernel(toks, emb, W, b):
    toks = toks.astype(jnp.int32)
    sidx = (toks[:, :, None] * SPT
            + jnp.arange(SPT, dtype=jnp.int32)).reshape(B, NSUB)
    sidx = jnp.pad(sidx, ((0, 0), (0, NPAD - NSUB)))
    emb3 = emb.reshape(D * 32000 // SUB, SUB)
    zsum_flat = _sc_segment_sum(sidx.reshape(-1), emb3)
    zsum = zsum_flat.reshape(B, D)
    return _tc_proj_norm(zsum, W, b.reshape(1, O))


# speedup vs baseline: 1.0353x; 1.0353x over previous
import functools

import jax
import jax.numpy as jnp
from jax import lax
from jax.experimental import pallas as pl
from jax.experimental.pallas import tpu as pltpu
from jax.experimental.pallas import tpu_sc as plsc

B = 4096
T = 50
D = 768
O = 1024
SUB = 256
SPT = D // SUB
NSUB = T * SPT
NPAD = 152
G1 = 104
G2 = NPAD - G1
NW = 32
BPW = B // NW
IDXW = BPW * NPAD


def _sc_segment_sum(sidx_flat, emb3):
    mesh = plsc.VectorSubcoreMesh(core_axis_name="c", subcore_axis_name="s")

    @functools.partial(
        pl.kernel,
        out_type=jax.ShapeDtypeStruct((B, D), jnp.float32),
        mesh=mesh,
        scratch_types=[
            pltpu.VMEM((IDXW,), jnp.int32),
            pltpu.VMEM((NPAD, SUB), jnp.float32),
            pltpu.VMEM((NPAD, SUB), jnp.float32),
            pltpu.VMEM((D,), jnp.float32),
            pltpu.SemaphoreType.DMA,
            pltpu.SemaphoreType.DMA,
        ],
    )
    def sc_kernel(sidx_hbm, emb_hbm, out_hbm, idx_v, rows0, rows1, stage,
                  sem0, sem1):
        wid = lax.axis_index("s") * 2 + lax.axis_index("c")
        rows = [rows0, rows1]
        sems = [sem0, sem1]
        base = wid * BPW

        pltpu.sync_copy(sidx_hbm.at[pl.ds(wid * IDXW, IDXW)], idx_v)

        def gather_copies(s, b):
            off = pl.multiple_of(b * NPAD, 8)
            return (
                pltpu.make_async_copy(
                    emb_hbm.at[idx_v.at[pl.ds(off, G1)]],
                    rows[s].at[pl.ds(0, G1)], sems[s]),
                pltpu.make_async_copy(
                    emb_hbm.at[idx_v.at[pl.ds(off + G1, G2)]],
                    rows[s].at[pl.ds(G1, G2)], sems[s]),
            )

        for s in range(2):
            for cp in gather_copies(s, s):
                cp.start()

        def step(g, carry):
            for s in range(2):
                b = g * 2 + s
                for cp in gather_copies(s, b):
                    cp.wait()

                for k in range(SPT):
                    def add_row(t, acc, k=k):
                        return tuple(
                            acc[v] + rows[s][t * SPT + k, pl.ds(v * 16, 16)]
                            for v in range(SUB // 16))

                    acc = lax.fori_loop(
                        0, T, add_row,
                        tuple(jnp.zeros((16,), jnp.float32)
                              for _ in range(SUB // 16)))
                    for v in range(SUB // 16):
                        stage[pl.ds(k * SUB + v * 16, 16)] = acc[v]

                @pl.when(b + 2 < BPW)
                def _():
                    for cp in gather_copies(s, b + 2):
                        cp.start()

                pltpu.sync_copy(stage, out_hbm.at[base + b])
            return carry

        lax.fori_loop(0, BPW // 2, step, 0)

    return sc_kernel(sidx_flat, emb3)


def _tc_proj_norm(zsum, W, b2d):
    blk = 256

    def tc_kernel(z_ref, w_ref, b_ref, o_ref):
        z = z_ref[...] * (1.0 / T)
        y = jnp.dot(z, w_ref[...], preferred_element_type=jnp.float32)
        y = y + b_ref[...]
        n = jnp.sqrt(jnp.sum(y * y, axis=1, keepdims=True))
        o_ref[...] = y / jnp.maximum(n, 1e-12)

    return pl.pallas_call(
        tc_kernel,
        grid=(B // blk,),
        in_specs=[
            pl.BlockSpec((blk, D), lambda i: (i, 0)),
            pl.BlockSpec((D, O), lambda i: (0, 0)),
            pl.BlockSpec((1, O), lambda i: (0, 0)),
        ],
        out_specs=pl.BlockSpec((blk, O), lambda i: (i, 0)),
        out_shape=jax.ShapeDtypeStruct((B, O), jnp.float32),
    )(zsum, W, b2d)


@jax.jit
def kernel(toks, emb, W, b):
    toks = toks.astype(jnp.int32)
    sidx = (toks[:, :, None] * SPT
            + jnp.arange(SPT, dtype=jnp.int32)).reshape(B, NSUB)
    sidx = jnp.pad(sidx, ((0, 0), (0, NPAD - NSUB)))
    emb3 = emb.reshape(D * 32000 // SUB, SUB)
    zsum = _sc_segment_sum(sidx.reshape(-1), emb3)
    return _tc_proj_norm(zsum, W, b.reshape(1, O))


# speedup vs baseline: 1.0445x; 1.0089x over previous
import functools

import jax
import jax.numpy as jnp
from jax import lax
from jax.experimental import pallas as pl
from jax.experimental.pallas import tpu as pltpu
from jax.experimental.pallas import tpu_sc as plsc

B = 4096
T = 50
D = 768
O = 1024
SUB = 256
SPT = D // SUB
NSUB = T * SPT
NPAD = 152
G1 = 104
G2 = NPAD - G1
NW = 32
BPW = B // NW
IDXW = BPW * NPAD


def _sc_segment_sum(sidx_flat, emb3):
    mesh = plsc.VectorSubcoreMesh(core_axis_name="c", subcore_axis_name="s")

    @functools.partial(
        pl.kernel,
        out_type=jax.ShapeDtypeStruct((B, D), jnp.float32),
        mesh=mesh,
        scratch_types=[
            pltpu.VMEM((IDXW,), jnp.int32),
            pltpu.VMEM((NPAD, SUB), jnp.float32),
            pltpu.VMEM((NPAD, SUB), jnp.float32),
            pltpu.VMEM((D,), jnp.float32),
            pltpu.SemaphoreType.DMA,
            pltpu.SemaphoreType.DMA,
        ],
    )
    def sc_kernel(sidx_hbm, emb_hbm, out_hbm, idx_v, rows0, rows1, stage,
                  sem0, sem1):
        wid = lax.axis_index("s") * 2 + lax.axis_index("c")
        rows = [rows0, rows1]
        sems = [sem0, sem1]
        base = wid * BPW

        pltpu.sync_copy(sidx_hbm.at[pl.ds(wid * IDXW, IDXW)], idx_v)

        def gather_copies(s, b):
            off = pl.multiple_of(b * NPAD, 8)
            return (
                pltpu.make_async_copy(
                    emb_hbm.at[idx_v.at[pl.ds(off, G1)]],
                    rows[s].at[pl.ds(0, G1)], sems[s]),
                pltpu.make_async_copy(
                    emb_hbm.at[idx_v.at[pl.ds(off + G1, G2)]],
                    rows[s].at[pl.ds(G1, G2)], sems[s]),
            )

        for s in range(2):
            for cp in gather_copies(s, s):
                cp.start()

        def step(g, carry):
            for s in range(2):
                b = g * 2 + s
                for cp in gather_copies(s, b):
                    cp.wait()

                for k in range(SPT):
                    def add_row(t, acc, k=k):
                        return tuple(
                            acc[v] + rows[s][t * SPT + k, pl.ds(v * 16, 16)]
                            for v in range(SUB // 16))

                    acc = lax.fori_loop(
                        0, T, add_row,
                        tuple(jnp.zeros((16,), jnp.float32)
                              for _ in range(SUB // 16)))
                    for v in range(SUB // 16):
                        stage[pl.ds(k * SUB + v * 16, 16)] = acc[v]

                @pl.when(b + 2 < BPW)
                def _():
                    for cp in gather_copies(s, b + 2):
                        cp.start()

                pltpu.sync_copy(stage, out_hbm.at[base + b])
            return carry

        lax.fori_loop(0, BPW // 2, step, 0)

    return sc_kernel(sidx_flat, emb3)


def _tc_proj_norm(zsum, W, b2d):
    blk = 512

    def tc_kernel(z_ref, w_ref, b_ref, o_ref):
        z = z_ref[...] * (1.0 / T)
        y = jnp.dot(z, w_ref[...], preferred_element_type=jnp.float32)
        y = y + b_ref[...]
        n = jnp.sqrt(jnp.sum(y * y, axis=1, keepdims=True))
        o_ref[...] = y / jnp.maximum(n, 1e-12)

    return pl.pallas_call(
        tc_kernel,
        grid=(B // blk,),
        in_specs=[
            pl.BlockSpec((blk, D), lambda i: (i, 0)),
            pl.BlockSpec((D, O), lambda i: (0, 0)),
            pl.BlockSpec((1, O), lambda i: (0, 0)),
        ],
        out_specs=pl.BlockSpec((blk, O), lambda i: (i, 0)),
        out_shape=jax.ShapeDtypeStruct((B, O), jnp.float32),
    )(zsum, W, b2d)


@jax.jit
def kernel(toks, emb, W, b):
    toks = toks.astype(jnp.int32)
    sidx = (toks[:, :, None] * SPT
            + jnp.arange(SPT, dtype=jnp.int32)).reshape(B, NSUB)
    sidx = jnp.pad(sidx, ((0, 0), (0, NPAD - NSUB)))
    emb3 = emb.reshape(D * 32000 // SUB, SUB)
    zsum = _sc_segment_sum(sidx.reshape(-1), emb3)
    return _tc_proj_norm(zsum, W, b.reshape(1, O))
